# Initial kernel scaffold; baseline (speedup 1.0000x reference)
import jax
import jax.numpy as jnp
from jax.experimental import pallas as pl


def kernel(tensor):
    raise NotImplementedError("write your pallas kernel here")


# speedup vs baseline: 1.4079x; 1.4079x over previous
import functools

import numpy as np
import jax
import jax.numpy as jnp
from jax.experimental import pallas as pl
from jax.experimental.pallas import tpu as pltpu

_TILE = 16
_IMG = 512
_STRIDE = 9
_NB = _IMG // _TILE
_NPATCH = _NB * _NB


_GPR = 8
_NGRP = _NPATCH // _GPR


def _gather_fold_body(sel_h_ref, sel_w_ref, tensor_ref, out_ref):
    lane = jax.lax.broadcasted_iota(jnp.int32, (_TILE, 128), 1)

    def group(g_all, _):
        bi = g_all // 4
        g = g_all % 4
        acc = jnp.zeros((_TILE, 128), jnp.float32)
        for p in range(_GPR):
            l = g_all * _GPR + p
            h = sel_h_ref[l]
            w = sel_w_ref[l]
            w0 = pl.multiple_of(jnp.minimum((w // 128) * 128, 2048 - 256), 128)
            d = w - w0
            h0 = pl.multiple_of((h // 8) * 8, 8)
            dh = h - h0
            window = tensor_ref[0, pl.ds(h0, 24), pl.ds(w0, 256)]
            window = pltpu.roll(window, (-dh) % 24, axis=0)[:_TILE]
            rolled = pltpu.roll(window, (_TILE * p - d) % 256, axis=1)
            acc = jnp.where(
                (lane >= _TILE * p) & (lane < _TILE * (p + 1)), rolled[:, :128], acc
            )
        out_ref[0, pl.ds(bi * _TILE, _TILE), pl.ds(pl.multiple_of(g * 128, 128), 128)] = acc
        return _

    jax.lax.fori_loop(0, _NGRP, group, 0)


def _gather_fold(tensor, sel_h, sel_w):
    grid_spec = pltpu.PrefetchScalarGridSpec(
        num_scalar_prefetch=2,
        grid=(3,),
        in_specs=[
            pl.BlockSpec((1, 2048, 2048), lambda c, sh, sw: (c, 0, 0)),
        ],
        out_specs=pl.BlockSpec((1, _IMG, _IMG), lambda c, sh, sw: (c, 0, 0)),
    )
    return pl.pallas_call(
        _gather_fold_body,
        grid_spec=grid_spec,
        out_shape=jax.ShapeDtypeStruct((3, _IMG, _IMG), jnp.float32),
    )(sel_h, sel_w, tensor)


def kernel(tensor):
    ts = _TILE
    C, H, W = tensor.shape
    w_lum = jnp.array([0.2989, 0.587, 0.114], dtype=tensor.dtype).reshape(3, 1, 1)
    gray = jnp.sum(w_lum * tensor, axis=0)
    gray = jnp.pad(gray, ((0, 0), (1, 1)))
    i1 = jnp.cumsum(jnp.cumsum(gray, axis=0), axis=1)
    i2 = jnp.cumsum(jnp.cumsum(gray ** 2, axis=0), axis=1)
    h_locs = np.arange(0, H - ts + 1, _STRIDE)
    w_locs = np.arange(0, W - ts + 1, _STRIDE)
    tl_h = jnp.asarray(np.repeat(h_locs, len(w_locs)), dtype=jnp.int32)
    tl_w = jnp.asarray(np.tile(w_locs, len(h_locs)), dtype=jnp.int32)
    br_h = tl_h + ts
    br_w = tl_w + ts
    sum1 = i1[br_h, br_w] + i1[tl_h, tl_w] - i1[tl_h, br_w] - i1[br_h, tl_w]
    sum2 = i2[br_h, br_w] + i2[tl_h, tl_w] - i2[tl_h, br_w] - i2[br_h, tl_w]
    n = ts * ts
    std_devs = jnp.sqrt((sum2 - sum1 ** 2 / n) / n)
    order = jnp.argsort(std_devs)[:_NPATCH]
    sel_h = tl_h[order]
    sel_w = tl_w[order]
    return _gather_fold(tensor, sel_h, sel_w)


# speedup vs baseline: 22.8226x; 16.2106x over previous
import functools

import numpy as np
import jax
import jax.numpy as jnp
from jax.experimental import pallas as pl
from jax.experimental.pallas import tpu as pltpu

_TILE = 16
_IMG = 512
_STRIDE = 9
_NB = _IMG // _TILE
_NPATCH = _NB * _NB


_GPR = 8
_NGRP = _NPATCH // _GPR


def _gather_fold_body(sel_h_ref, sel_w_ref, tensor_ref, out_ref):
    lane = jax.lax.broadcasted_iota(jnp.int32, (_TILE, 128), 1)

    def group(g_all, _):
        bi = g_all // 4
        g = g_all % 4
        acc = jnp.zeros((_TILE, 128), jnp.float32)
        for p in range(_GPR):
            l = g_all * _GPR + p
            h = sel_h_ref[l]
            w = sel_w_ref[l]
            w0 = pl.multiple_of(jnp.minimum((w // 128) * 128, 2048 - 256), 128)
            d = w - w0
            h0 = pl.multiple_of((h // 8) * 8, 8)
            dh = h - h0
            window = tensor_ref[0, pl.ds(h0, 24), pl.ds(w0, 256)]
            window = pltpu.roll(window, (-dh) % 24, axis=0)[:_TILE]
            rolled = pltpu.roll(window, (_TILE * p - d) % 256, axis=1)
            acc = jnp.where(
                (lane >= _TILE * p) & (lane < _TILE * (p + 1)), rolled[:, :128], acc
            )
        out_ref[0, pl.ds(bi * _TILE, _TILE), pl.ds(pl.multiple_of(g * 128, 128), 128)] = acc
        return _

    jax.lax.fori_loop(0, _NGRP, group, 0)


def _gather_fold(tensor, sel_h, sel_w):
    grid_spec = pltpu.PrefetchScalarGridSpec(
        num_scalar_prefetch=2,
        grid=(3,),
        in_specs=[
            pl.BlockSpec((1, 2048, 2048), lambda c, sh, sw: (c, 0, 0)),
        ],
        out_specs=pl.BlockSpec((1, _IMG, _IMG), lambda c, sh, sw: (c, 0, 0)),
    )
    return pl.pallas_call(
        _gather_fold_body,
        grid_spec=grid_spec,
        out_shape=jax.ShapeDtypeStruct((3, _IMG, _IMG), jnp.float32),
    )(sel_h, sel_w, tensor)


def kernel(tensor):
    ts = _TILE
    C, H, W = tensor.shape
    w_lum = jnp.array([0.2989, 0.587, 0.114], dtype=tensor.dtype).reshape(3, 1, 1)
    gray = jnp.sum(w_lum * tensor, axis=0)
    gray = jnp.pad(gray, ((0, 0), (1, 1)))
    i1 = jnp.cumsum(jnp.cumsum(gray, axis=0), axis=1)
    i2 = jnp.cumsum(jnp.cumsum(gray ** 2, axis=0), axis=1)
    ng = (H - ts) // _STRIDE + 1
    lim = (ng - 1) * _STRIDE + 1
    tl = i1[0:lim:_STRIDE, 0:lim:_STRIDE]
    br = i1[ts:ts + lim:_STRIDE, ts:ts + lim:_STRIDE]
    tr = i1[0:lim:_STRIDE, ts:ts + lim:_STRIDE]
    bl = i1[ts:ts + lim:_STRIDE, 0:lim:_STRIDE]
    tl2 = i2[0:lim:_STRIDE, 0:lim:_STRIDE]
    br2 = i2[ts:ts + lim:_STRIDE, ts:ts + lim:_STRIDE]
    tr2 = i2[0:lim:_STRIDE, ts:ts + lim:_STRIDE]
    bl2 = i2[ts:ts + lim:_STRIDE, 0:lim:_STRIDE]
    sum1 = (br + tl - tr - bl).reshape(-1)
    sum2 = (br2 + tl2 - tr2 - bl2).reshape(-1)
    n = ts * ts
    std_devs = jnp.sqrt((sum2 - sum1 ** 2 / n) / n)
    order = jnp.argsort(std_devs)[:_NPATCH]
    sel_h = (order // ng) * _STRIDE
    sel_w = (order % ng) * _STRIDE
    return _gather_fold(tensor, sel_h.astype(jnp.int32), sel_w.astype(jnp.int32))
